# baseline (speedup 1.0000x reference)
import functools

import jax
import jax.numpy as jnp
from jax import lax
from jax.experimental import pallas as pl
from jax.experimental.pallas import tpu as pltpu
from jax.experimental.pallas import tpu_sc as plsc

H = 1024
W = 1024
M = H * W
N = 1048576
NC, NS = 2, 16
NW = NC * NS
T = N // NW
P = 4096
PIECES = T // P
SC_CHUNK = M // NS
GB = 4


def _sc_body(ob_hbm, hist_hbm, cnt_hbm, a_hbm, b_hbm,
             obr, obc, idxb, cnt, ones, shared, sob, sga, ssc, sout):
    c = lax.axis_index("c")
    s = lax.axis_index("s")
    wid = s * NC + c

    def _fill(i, carry):
        ones[pl.ds(i * 16, 16)] = jnp.full((16,), 1.0, jnp.float32)
        cnt[0][pl.ds(i * 16, 16)] = jnp.zeros((16,), jnp.float32)
        return carry
    lax.fori_loop(0, P // 16, _fill, 0)

    @pl.when(c == 0)
    def _():
        d = pltpu.async_copy(hist_hbm.at[pl.ds(s * SC_CHUNK, SC_CHUNK)],
                             shared.at[pl.ds(s * SC_CHUNK, SC_CHUNK)], sout)
        d.wait()

    @pl.when(c == 1)
    def _():
        zdma = [pltpu.async_copy(cnt[0],
                                 shared.at[pl.ds(s * SC_CHUNK + k * P, P)],
                                 sout)
                for k in range(SC_CHUNK // P)]
        for d in zdma:
            d.wait()
    plsc.subcore_barrier()

    def _start_obs(p, b):
        off = wid * T + p * P
        return (pltpu.async_copy(ob_hbm.at[pl.ds(off, P)], obr[b], sob),
                pltpu.async_copy(ob_hbm.at[pl.ds(N + off, P)], obc[b], sob))

    def _idx_loop(ob, j):
        def _mkidx(i, carry):
            r = obr[ob][pl.ds(i * 16, 16)]
            q = obc[ob][pl.ds(i * 16, 16)]
            ri = jnp.minimum(jnp.maximum(r.astype(jnp.int32), 0), H - 1)
            ci = jnp.minimum(jnp.maximum(q.astype(jnp.int32), 0), W - 1)
            idxb[j][pl.ds(i * 16, 16)] = ri * W + ci
            return carry
        lax.fori_loop(0, P // 16, _mkidx, 0)

    def _gather(p):
        j = p % 3
        b = p % 2
        g = pltpu.async_copy(hist_hbm.at[idxb[j]], cnt[b], sga)
        sc = pltpu.async_copy(ones, shared.at[idxb[j]], ssc, add=True)
        return g, sc

    obs_dma = [_start_obs(0, 0), _start_obs(1, 1)]
    gat = [None, None]
    sc_dma = [None, None, None]
    out_dma = [None, None]
    for d in obs_dma[0]:
        d.wait()
    _idx_loop(0, 0)
    if 2 < PIECES:
        obs_dma[0] = _start_obs(2, 0)
    gat[0], sc_dma[0] = _gather(0)
    for p in range(PIECES):
        if p + 1 < PIECES:
            j1 = (p + 1) % 3
            b1 = (p + 1) % 2
            if sc_dma[j1] is not None:
                sc_dma[j1].wait()
            for d in obs_dma[b1]:
                d.wait()
            _idx_loop(b1, j1)
            if p + 3 < PIECES:
                obs_dma[b1] = _start_obs(p + 3, b1)
            if out_dma[b1] is not None:
                out_dma[b1].wait()
            gat[b1], sc_dma[j1] = _gather(p + 1)
        b = p % 2
        gat[b].wait()
        out_dma[b] = pltpu.async_copy(
            cnt[b], cnt_hbm.at[pl.ds(wid * T + p * P, P)], sout)
    for d in out_dma:
        if d is not None:
            d.wait()
    for d in sc_dma:
        if d is not None:
            d.wait()
    plsc.subcore_barrier()

    @pl.when(c == 0)
    def _():
        pltpu.sync_copy(shared.at[pl.ds(s * SC_CHUNK, SC_CHUNK)],
                        a_hbm.at[pl.ds(s * SC_CHUNK, SC_CHUNK)])

    @pl.when(c == 1)
    def _():
        pltpu.sync_copy(shared.at[pl.ds(s * SC_CHUNK, SC_CHUNK)],
                        b_hbm.at[pl.ds(s * SC_CHUNK, SC_CHUNK)])


_sc_call = functools.partial(
    pl.kernel,
    out_type=(
        jax.ShapeDtypeStruct((N,), jnp.float32),
        jax.ShapeDtypeStruct((M,), jnp.float32),
        jax.ShapeDtypeStruct((M,), jnp.float32),
    ),
    mesh=plsc.VectorSubcoreMesh(core_axis_name="c", subcore_axis_name="s"),
    scratch_types=[
        [pltpu.VMEM((P,), jnp.float32)] * 2,
        [pltpu.VMEM((P,), jnp.float32)] * 2,
        [pltpu.VMEM((P,), jnp.int32)] * 3,
        [pltpu.VMEM((P,), jnp.float32)] * 2,
        pltpu.VMEM((P,), jnp.float32),
        pltpu.VMEM_SHARED((M,), jnp.float32),
        pltpu.SemaphoreType.DMA,
        pltpu.SemaphoreType.DMA,
        pltpu.SemaphoreType.DMA,
        pltpu.SemaphoreType.DMA,
    ],
)(_sc_body)


def _combine_body(n_ref, a_ref, b_ref, cnt_ref, oh_ref, ob_ref):
    s2 = 2.0 * jnp.log(jnp.float32(n_ref[0]) + jnp.float32(N))
    oh_ref[...] = (a_ref[...] + b_ref[...]).reshape(H // GB, W)
    ob_ref[...] = jnp.sqrt(s2 / (cnt_ref[...] + 1.0))


_combine = pl.pallas_call(
    _combine_body,
    grid=(GB,),
    in_specs=[
        pl.BlockSpec(memory_space=pltpu.MemorySpace.SMEM),
        pl.BlockSpec((M // GB,), lambda i: (i,)),
        pl.BlockSpec((M // GB,), lambda i: (i,)),
        pl.BlockSpec((N // GB,), lambda i: (i,)),
    ],
    out_specs=[
        pl.BlockSpec((H // GB, W), lambda i: (i, 0)),
        pl.BlockSpec((N // GB,), lambda i: (i,)),
    ],
    out_shape=(
        jax.ShapeDtypeStruct((H, W), jnp.float32),
        jax.ShapeDtypeStruct((N,), jnp.float32),
    ),
)


def kernel(ob_no, histogram, n):
    n_arr = jnp.reshape(jnp.asarray(n, jnp.int32), (1,))
    cnt, part_a, part_b = _sc_call(
        ob_no.T.reshape(-1), histogram.reshape(-1))
    new_hist, bonus = _combine(n_arr, part_a, part_b, cnt)
    return bonus, new_hist


# speedup vs baseline: 1.0241x; 1.0241x over previous
import functools

import jax
import jax.numpy as jnp
from jax import lax
from jax.experimental import pallas as pl
from jax.experimental.pallas import tpu as pltpu
from jax.experimental.pallas import tpu_sc as plsc

H = 1024
W = 1024
M = H * W
N = 1048576
NC, NS = 2, 16
NW = NC * NS
T = N // NW
P = 4096
PIECES = T // P
SC_CHUNK = M // NS
GB = 4


def _sc_body(obr_hbm, obc_hbm, hist_hbm, cnt_hbm, a_hbm, b_hbm,
             obr, obc, idxb, cnt, ones, shared, sob, sga, ssc, sout):
    c = lax.axis_index("c")
    s = lax.axis_index("s")
    wid = s * NC + c

    def _fill(i, carry):
        ones[pl.ds(i * 16, 16)] = jnp.full((16,), 1.0, jnp.float32)
        cnt[0][pl.ds(i * 16, 16)] = jnp.zeros((16,), jnp.float32)
        return carry
    lax.fori_loop(0, P // 16, _fill, 0)

    @pl.when(c == 0)
    def _():
        d = pltpu.async_copy(hist_hbm.at[pl.ds(s * SC_CHUNK, SC_CHUNK)],
                             shared.at[pl.ds(s * SC_CHUNK, SC_CHUNK)], sout)
        d.wait()

    @pl.when(c == 1)
    def _():
        zdma = [pltpu.async_copy(cnt[0],
                                 shared.at[pl.ds(s * SC_CHUNK + k * P, P)],
                                 sout)
                for k in range(SC_CHUNK // P)]
        for d in zdma:
            d.wait()
    plsc.subcore_barrier()

    def _start_obs(p, b):
        off = wid * T + p * P
        return (pltpu.async_copy(obr_hbm.at[pl.ds(off, P)], obr[b], sob),
                pltpu.async_copy(obc_hbm.at[pl.ds(off, P)], obc[b], sob))

    def _idx_loop(ob, j):
        def _mkidx(i, carry):
            r = obr[ob][pl.ds(i * 16, 16)]
            q = obc[ob][pl.ds(i * 16, 16)]
            ri = jnp.minimum(jnp.maximum(r.astype(jnp.int32), 0), H - 1)
            ci = jnp.minimum(jnp.maximum(q.astype(jnp.int32), 0), W - 1)
            idxb[j][pl.ds(i * 16, 16)] = ri * W + ci
            return carry
        lax.fori_loop(0, P // 16, _mkidx, 0)

    def _gather(p):
        j = p % 3
        b = p % 2
        g = pltpu.async_copy(hist_hbm.at[idxb[j]], cnt[b], sga)
        sc = pltpu.async_copy(ones, shared.at[idxb[j]], ssc, add=True)
        return g, sc

    obs_dma = [_start_obs(0, 0), _start_obs(1, 1)]
    gat = [None, None]
    sc_dma = [None, None, None]
    out_dma = [None, None]
    for d in obs_dma[0]:
        d.wait()
    _idx_loop(0, 0)
    if 2 < PIECES:
        obs_dma[0] = _start_obs(2, 0)
    gat[0], sc_dma[0] = _gather(0)
    for p in range(PIECES):
        if p + 1 < PIECES:
            j1 = (p + 1) % 3
            b1 = (p + 1) % 2
            if sc_dma[j1] is not None:
                sc_dma[j1].wait()
            for d in obs_dma[b1]:
                d.wait()
            _idx_loop(b1, j1)
            if p + 3 < PIECES:
                obs_dma[b1] = _start_obs(p + 3, b1)
            if out_dma[b1] is not None:
                out_dma[b1].wait()
            gat[b1], sc_dma[j1] = _gather(p + 1)
        b = p % 2
        gat[b].wait()
        out_dma[b] = pltpu.async_copy(
            cnt[b], cnt_hbm.at[pl.ds(wid * T + p * P, P)], sout)
    for d in out_dma:
        if d is not None:
            d.wait()
    for d in sc_dma:
        if d is not None:
            d.wait()
    plsc.subcore_barrier()

    @pl.when(c == 0)
    def _():
        pltpu.sync_copy(shared.at[pl.ds(s * SC_CHUNK, SC_CHUNK)],
                        a_hbm.at[pl.ds(s * SC_CHUNK, SC_CHUNK)])

    @pl.when(c == 1)
    def _():
        pltpu.sync_copy(shared.at[pl.ds(s * SC_CHUNK, SC_CHUNK)],
                        b_hbm.at[pl.ds(s * SC_CHUNK, SC_CHUNK)])


_sc_call = functools.partial(
    pl.kernel,
    out_type=(
        jax.ShapeDtypeStruct((N,), jnp.float32),
        jax.ShapeDtypeStruct((M,), jnp.float32),
        jax.ShapeDtypeStruct((M,), jnp.float32),
    ),
    mesh=plsc.VectorSubcoreMesh(core_axis_name="c", subcore_axis_name="s"),
    scratch_types=[
        [pltpu.VMEM((P,), jnp.float32)] * 2,
        [pltpu.VMEM((P,), jnp.float32)] * 2,
        [pltpu.VMEM((P,), jnp.int32)] * 3,
        [pltpu.VMEM((P,), jnp.float32)] * 2,
        pltpu.VMEM((P,), jnp.float32),
        pltpu.VMEM_SHARED((M,), jnp.float32),
        pltpu.SemaphoreType.DMA,
        pltpu.SemaphoreType.DMA,
        pltpu.SemaphoreType.DMA,
        pltpu.SemaphoreType.DMA,
    ],
)(_sc_body)


def _combine_body(n_ref, a_ref, b_ref, cnt_ref, oh_ref, ob_ref):
    s2 = 2.0 * jnp.log(jnp.float32(n_ref[0]) + jnp.float32(N))
    oh_ref[...] = (a_ref[...] + b_ref[...]).reshape(H // GB, W)
    ob_ref[...] = jnp.sqrt(s2 / (cnt_ref[...] + 1.0))


_combine = pl.pallas_call(
    _combine_body,
    grid=(GB,),
    in_specs=[
        pl.BlockSpec(memory_space=pltpu.MemorySpace.SMEM),
        pl.BlockSpec((M // GB,), lambda i: (i,)),
        pl.BlockSpec((M // GB,), lambda i: (i,)),
        pl.BlockSpec((N // GB,), lambda i: (i,)),
    ],
    out_specs=[
        pl.BlockSpec((H // GB, W), lambda i: (i, 0)),
        pl.BlockSpec((N // GB,), lambda i: (i,)),
    ],
    out_shape=(
        jax.ShapeDtypeStruct((H, W), jnp.float32),
        jax.ShapeDtypeStruct((N,), jnp.float32),
    ),
)


def kernel(ob_no, histogram, n):
    n_arr = jnp.reshape(jnp.asarray(n, jnp.int32), (1,))
    cnt, part_a, part_b = _sc_call(
        ob_no[:, 0], ob_no[:, 1], histogram.reshape(-1))
    new_hist, bonus = _combine(n_arr, part_a, part_b, cnt)
    return bonus, new_hist
